# baseline (speedup 1.0000x reference)
import functools

import jax
import jax.numpy as jnp
from jax import lax
from jax.experimental import pallas as pl
from jax.experimental.pallas import tpu as pltpu
from jax.experimental.pallas import tpu_sc as plsc

_NBUF = 2


def _make_gather(B, F, V, D, num_cores, num_subcores):
    NW = num_cores * num_subcores
    b_per_w = B // NW
    NI = b_per_w * F
    NB = 8
    CH = NB * F
    CP = 256
    n_ch = b_per_w // NB
    NG = CH // 16
    D2 = 2 * D
    splits = []
    off = 0
    while off < CH:
        w = min(128, CH - off)
        splits.append((off, w))
        off += w
    assert n_ch % _NBUF == 0
    mesh = plsc.VectorSubcoreMesh(core_axis_name="c", subcore_axis_name="s")

    @functools.partial(
        pl.kernel,
        out_type=jax.ShapeDtypeStruct((B, F * D), jnp.float32),
        mesh=mesh,
        scratch_types=[
            pltpu.VMEM((NI,), jnp.int32),
            pltpu.VMEM((n_ch * CP,), jnp.int32),
            pltpu.VMEM((_NBUF, CH, D2), jnp.float32),
            pltpu.VMEM((NB, F * D), jnp.float32),
            [pltpu.SemaphoreType.DMA] * _NBUF,
        ],
        compiler_params=pltpu.CompilerParams(use_tc_tiling_on_sc=True),
    )
    def gather_kernel(idx_hbm, tbl_hbm, out_hbm, idx_v, pair_v, rows_v,
                      packed_v, gsems):
        wid = lax.axis_index("s") * num_cores + lax.axis_index("c")
        b0 = wid * b_per_w
        one_v = jnp.full((16,), 1, jnp.int32)

        pltpu.sync_copy(idx_hbm.at[pl.ds(wid * NI, NI)], idx_v)

        def shift_body(i, carry):
            def g_body(g, c2):
                iv = idx_v[pl.ds(i * CH + g * 16, 16)]
                pair_v[pl.ds(i * CP + g * 16, 16)] = (
                    jax.lax.shift_right_logical(iv, one_v)
                )
                return c2

            lax.fori_loop(0, NG, g_body, 0)
            return carry

        lax.fori_loop(0, n_ch, shift_body, 0)

        def fire_gather(i, b):
            for (o, w) in splits:
                pltpu.async_copy(
                    tbl_hbm.at[pair_v.at[pl.ds(i * CP + o, w)]],
                    rows_v.at[b].at[pl.ds(o, w)],
                    gsems[b],
                )

        def wait_gather(i, b):
            for (o, w) in splits:
                pltpu.make_async_copy(
                    tbl_hbm.at[pair_v.at[pl.ds(i * CP + o, w)]],
                    rows_v.at[b].at[pl.ds(o, w)],
                    gsems[b],
                ).wait()

        def compact(i, b):
            def group_body(g, carry):
                par16 = jnp.bitwise_and(idx_v[pl.ds(i * CH + g * 16, 16)],
                                        one_v)
                for l in range(16):
                    k = g * 16 + l
                    bb = k // F
                    f = k % F
                    src0 = par16[l] * D
                    dst0 = f * D
                    for c in range(D // 16):
                        packed_v[bb, pl.ds(dst0 + c * 16, 16)] = rows_v[
                            b, k, pl.ds(src0 + c * 16, 16)
                        ]
                return carry

            lax.fori_loop(0, NG, group_body, 0)

        def store_out(i):
            pltpu.sync_copy(packed_v, out_hbm.at[pl.ds(b0 + i * NB, NB)])

        for b in range(_NBUF):
            fire_gather(b, b)

        def body(it, carry):
            i0 = it * _NBUF
            for b in range(_NBUF):
                i = i0 + b
                wait_gather(i, b)
                compact(i, b)
                store_out(i)

                @pl.when(i + _NBUF < n_ch)
                def _():
                    fire_gather(i + _NBUF, b)
            return carry

        lax.fori_loop(0, n_ch // _NBUF, body, 0)

    return gather_kernel


def kernel(x, table):
    B, F = x.shape
    V, D = table.shape
    flat_idx = x.reshape(B * F).astype(jnp.int32)
    tbl2 = table.reshape(V // 2, 2 * D)
    gather = _make_gather(B, F, V, D, 2, 16)
    return gather(flat_idx, tbl2)
